# baseline (speedup 1.0000x reference)
import jax
import jax.numpy as jnp
import numpy as np
from jax.experimental import pallas as pl
from jax.experimental.pallas import tpu as pltpu

_STEPS = 24
_A = 8
_H = 1024
_TR = 256
_CPM = 4 * _H // _TR
_GRID = 4 * _CPM + 1
_F32 = jnp.float32
_BF16 = jnp.bfloat16
_HI = jax.lax.Precision.HIGHEST


def _gumbel_table():
    skey = jax.random.key(42)
    rows = [
        jax.random.gumbel(jax.random.fold_in(skey, s), (1, _A), _F32)
        for s in range(_STEPS)
    ]
    return jnp.concatenate(rows, axis=0)


def _body(g_ref, emb_ref, softt_ref, wih_ref, whh_ref, bih_ref, bhh_ref,
          gum_ref, stats_ref, arch_ref, w0n, w1xn, w1hn, es9, ls):
    H = _H
    i = pl.program_id(0)

    def prologue(start, work):
        @pl.when((i >= start) & (i < start + _CPM))
        def _():
            work(i - start)
        return None

    def e_work(j):
        rows = jnp.concatenate([g_ref[...], emb_ref[...]], axis=0)
        part = jax.lax.dot_general(
            wih_ref[0], rows.T, (((1,), (0,)), ((), ())),
            preferred_element_type=_F32, precision=_HI)
        es9[0:9, j * _TR:(j + 1) * _TR] = part.T

    def split_work(ref, out_ref):
        def work(j):
            block = ref[0]
            hi = block.astype(_BF16)
            lo = (block - hi.astype(_F32)).astype(_BF16)
            out_ref[pl.ds(j * _TR, _TR), 0:H] = hi
            out_ref[pl.ds(j * _TR, _TR), H:2 * H] = lo
        return work

    for jj in range(_CPM):
        @pl.when(i == jj)
        def _(jj=jj):
            e_work(jj)
    prologue(1 * _CPM, split_work(whh_ref, w0n))
    prologue(2 * _CPM, split_work(wih_ref, w1xn))
    prologue(3 * _CPM, split_work(whh_ref, w1hn))

    @pl.when(i == _GRID - 1)
    def _():
        iota_a = jax.lax.broadcasted_iota(jnp.int32, (1, _A), 1)
        iota_t = jax.lax.broadcasted_iota(jnp.int32, (1, _STEPS), 1)
        b0 = bih_ref[0:1, :] + bhh_ref[0:1, :]
        b1 = bih_ref[1:2, :] + bhh_ref[1:2, :]
        soft = softt_ref[...].T
        zcol = jnp.zeros((H, 1), _BF16)

        def hilo_cols(x):
            hi = x.astype(_BF16)
            lo = (x - hi.astype(_F32)).astype(_BF16)
            hc, lc = hi.T, lo.T
            return jnp.concatenate([
                jnp.concatenate([hc, lc], axis=1),
                jnp.concatenate([hc, zcol], axis=1)], axis=0)

        def cell(gates, c):
            i_g = gates[:, 0:H]
            f_g = gates[:, H:2 * H]
            g_g = gates[:, 2 * H:3 * H]
            o_g = gates[:, 3 * H:4 * H]
            c_new = (jax.nn.sigmoid(f_g) * c
                     + jax.nn.sigmoid(i_g) * jnp.tanh(g_g))
            h_new = jax.nn.sigmoid(o_g) * jnp.tanh(c_new)
            return h_new, c_new

        def wdot(w_ref, xcols):
            r = jax.lax.dot_general(
                w_ref[...], xcols, (((1,), (0,)), ((), ())),
                preferred_element_type=_F32)
            return (r[:, 0:1] + r[:, 1:2]).T

        def step_fn(t, carry):
            x0e, d0, c0, d1h, c1, act_row = carry
            g0 = x0e + d0 + b0
            h0n, c0n = cell(g0, c0)
            a0n = hilo_cols(h0n)
            d0n = wdot(w0n, a0n)
            d1x = wdot(w1xn, a0n)
            g1 = d1x + d1h + b1
            h1n, c1n = cell(g1, c1)
            d1hn = wdot(w1hn, hilo_cols(h1n))
            logits = jax.lax.dot_general(
                h1n, soft, (((1,), (0,)), ((), ())),
                preferred_element_type=_F32, precision=_HI)
            ls[pl.ds(t, 1), :] = logits
            z = logits + gum_ref[pl.ds(t, 1), :]
            a = jnp.min(
                jnp.where(z >= jnp.max(z), iota_a, _A)).astype(jnp.int32)
            x0e_next = es9[pl.ds(a + 1, 1), :]
            act_row = jnp.where(iota_t == t, a, act_row)
            return (x0e_next, d0n, c0n, d1hn, c1n, act_row)

        zvec = jnp.zeros((1, 4 * H), _F32)
        init = (es9[0:1, :], zvec, jnp.zeros((1, H), _F32),
                zvec, jnp.zeros((1, H), _F32),
                jnp.zeros((1, _STEPS), jnp.int32))
        carry = jax.lax.fori_loop(0, _STEPS, step_fn, init)
        act_row = carry[5]

        L = ls[...]
        m = jnp.max(L, axis=1, keepdims=True)
        logp = L - (m + jnp.log(jnp.sum(jnp.exp(L - m), axis=1,
                                        keepdims=True)))
        ent_col = -jnp.sum(jnp.exp(logp) * logp, axis=1, keepdims=True)
        oh = (jax.lax.broadcasted_iota(jnp.int32, (_STEPS, _A), 1)
              == act_row.T)
        lp_col = jnp.sum(jnp.where(oh, logp, 0.0), axis=1, keepdims=True)
        stats_ref[0:1, :] = lp_col.T
        stats_ref[1:2, :] = ent_col.T
        arch_ref[...] = act_row


def _wih_index(i):
    l = jnp.where(i < 2 * _CPM, 0, 1)
    j = jnp.clip(jnp.where(i < 2 * _CPM, i, i - 2 * _CPM), 0, _CPM - 1)
    return (l, j, 0)


def _whh_index(i):
    l = jnp.where(i < 3 * _CPM, 0, 1)
    j = jnp.clip(jnp.where(i < 3 * _CPM, i - _CPM, i - 3 * _CPM),
                 0, _CPM - 1)
    return (l, j, 0)


def _full(shape):
    return pl.BlockSpec(shape, lambda i: tuple(0 for _ in shape))


def kernel(g_emb, w_emb, soft_emb, W_ih, W_hh, b_ih, b_hh):
    gum = _gumbel_table()
    stats, arch_row = pl.pallas_call(
        _body,
        grid=(_GRID,),
        in_specs=[
            _full((1, _H)),
            _full((_A, _H)),
            _full((_A, _H)),
            pl.BlockSpec((1, _TR, _H), _wih_index),
            pl.BlockSpec((1, _TR, _H), _whh_index),
            _full((2, 4 * _H)),
            _full((2, 4 * _H)),
            _full((_STEPS, _A)),
        ],
        out_specs=[
            _full((2, _STEPS)),
            _full((1, _STEPS)),
        ],
        out_shape=[
            jax.ShapeDtypeStruct((2, _STEPS), _F32),
            jax.ShapeDtypeStruct((1, _STEPS), jnp.int32),
        ],
        scratch_shapes=[
            pltpu.VMEM((4 * _H, 2 * _H), _BF16),
            pltpu.VMEM((4 * _H, 2 * _H), _BF16),
            pltpu.VMEM((4 * _H, 2 * _H), _BF16),
            pltpu.VMEM((16, 4 * _H), _F32),
            pltpu.VMEM((_STEPS, _A), _F32),
        ],
        compiler_params=pltpu.CompilerParams(
            dimension_semantics=("arbitrary",),
            vmem_limit_bytes=63 * 1024 * 1024),
    )(g_emb, w_emb, soft_emb.T, W_ih, W_hh, b_ih, b_hh, gum)
    return stats, arch_row[0]


# speedup vs baseline: 1.9945x; 1.9945x over previous
import jax
import jax.numpy as jnp
import numpy as np
from jax.experimental import pallas as pl
from jax.experimental.pallas import tpu as pltpu

_STEPS = 24
_A = 8
_H = 1024
_TW = 128
_TH = 2 * _H
_TPH = _H // _TW
_TPM = 2 * _TPH
_GRID = 4 * _TPM + 1
_F32 = jnp.float32
_BF16 = jnp.bfloat16
_HI = jax.lax.Precision.HIGHEST


def _gumbel_table():
    skey = jax.random.key(42)
    rows = [
        jax.random.gumbel(jax.random.fold_in(skey, s), (1, _A), _F32)
        for s in range(_STEPS)
    ]
    return jnp.concatenate(rows, axis=0)


def _body(g_ref, emb_ref, softt_ref, wih_ref, whh_ref, bih_ref, bhh_ref,
          gum_ref, stats_ref, arch_ref, w0s, w1s, es, ls):
    H = _H
    i = pl.program_id(0)

    def prologue(start, half, work):
        @pl.when((i >= start) & (i < start + _TPH))
        def _():
            j = i - start
            work(j * _TW, j, half * _TH)

    def e_work(ref):
        def work(row_base, j, col0):
            rows_tile = jnp.concatenate(
                [g_ref[:, pl.ds(row_base, _TW)],
                 emb_ref[:, pl.ds(row_base, _TW)]], axis=0)
            part = jax.lax.dot_general(
                ref[0], rows_tile.T, (((1,), (0,)), ((), ())),
                preferred_element_type=_F32, precision=_HI)
            prev = jnp.where(j == 0, jnp.zeros((9, _TH), _F32),
                             es[:, col0:col0 + _TH])
            es[:, col0:col0 + _TH] = prev + part.T
        return work

    def split_work(ref, out_ref, base):
        def work(row_base, j, col0):
            bt = ref[0].T
            hi = bt.astype(_BF16)
            lo = (bt - hi.astype(_F32)).astype(_BF16)
            out_ref[pl.ds(base + row_base, _TW), col0:col0 + _TH] = hi
            out_ref[pl.ds(base + H + row_base, _TW), col0:col0 + _TH] = lo
        return work

    for half in (0, 1):
        prologue(0 * _TPM + half * _TPH, half, e_work(wih_ref))
        prologue(1 * _TPM + half * _TPH, half, split_work(whh_ref, w0s, 0))
        prologue(2 * _TPM + half * _TPH, half, split_work(wih_ref, w1s, 0))
        prologue(3 * _TPM + half * _TPH, half,
                 split_work(whh_ref, w1s, 2 * H))

    @pl.when(i == _GRID - 1)
    def _():
        iota_a = jax.lax.broadcasted_iota(jnp.int32, (1, _A), 1)
        iota_t = jax.lax.broadcasted_iota(jnp.int32, (1, _STEPS), 1)
        b0 = bih_ref[0:1, :] + bhh_ref[0:1, :]
        b1 = bih_ref[1:2, :] + bhh_ref[1:2, :]
        soft = softt_ref[...].T

        def hilo2(x):
            hi = x.astype(_BF16)
            lo = (x - hi.astype(_F32)).astype(_BF16)
            return jnp.concatenate([
                jnp.concatenate([hi, hi], axis=1),
                jnp.concatenate([lo, jnp.zeros((1, H), _BF16)], axis=1)],
                axis=0)

        def cell(gates, c):
            i_g = gates[:, 0:H]
            f_g = gates[:, H:2 * H]
            g_g = gates[:, 2 * H:3 * H]
            o_g = gates[:, 3 * H:4 * H]
            c_new = (jax.nn.sigmoid(f_g) * c
                     + jax.nn.sigmoid(i_g) * jnp.tanh(g_g))
            h_new = jax.nn.sigmoid(o_g) * jnp.tanh(c_new)
            return h_new, c_new

        def bdot(act, wv):
            r = jax.lax.dot_general(
                act, wv, (((1,), (0,)), ((), ())),
                preferred_element_type=_F32)
            return r[0:1, :] + r[1:2, :]

        def step_fn(t, carry):
            x0e, d0, c0, d1h, c1, act_row = carry
            g0 = x0e + d0 + b0
            h0n, c0n = cell(g0, c0)
            a0n = hilo2(h0n)
            d0n = bdot(a0n, w0s[...])
            d1x = bdot(a0n, w1s[0:2 * H, :])
            g1 = d1x + d1h + b1
            h1n, c1n = cell(g1, c1)
            d1hn = bdot(hilo2(h1n), w1s[2 * H:4 * H, :])
            logits = jax.lax.dot_general(
                h1n, soft, (((1,), (0,)), ((), ())),
                preferred_element_type=_F32, precision=_HI)
            ls[pl.ds(t, 1), :] = logits
            z = logits + gum_ref[pl.ds(t, 1), :]
            a = jnp.min(
                jnp.where(z >= jnp.max(z), iota_a, _A)).astype(jnp.int32)
            x0e_next = es[pl.ds(a + 1, 1), :]
            act_row = jnp.where(iota_t == t, a, act_row)
            return (x0e_next, d0n, c0n, d1hn, c1n, act_row)

        zvec = jnp.zeros((1, 4 * H), _F32)
        init = (es[0:1, :], zvec, jnp.zeros((1, H), _F32),
                zvec, jnp.zeros((1, H), _F32),
                jnp.zeros((1, _STEPS), jnp.int32))
        carry = jax.lax.fori_loop(0, _STEPS, step_fn, init)
        act_row = carry[5]

        L = ls[...]
        m = jnp.max(L, axis=1, keepdims=True)
        logp = L - (m + jnp.log(jnp.sum(jnp.exp(L - m), axis=1,
                                        keepdims=True)))
        ent_col = -jnp.sum(jnp.exp(logp) * logp, axis=1, keepdims=True)
        oh = (jax.lax.broadcasted_iota(jnp.int32, (_STEPS, _A), 1)
              == act_row.T)
        lp_col = jnp.sum(jnp.where(oh, logp, 0.0), axis=1, keepdims=True)
        stats_ref[0:1, :] = lp_col.T
        stats_ref[1:2, :] = ent_col.T
        arch_ref[...] = act_row


def _wih_index(i):
    l = jnp.where(i < 2 * _TPM, 0, 1)
    s = jnp.clip(jnp.where(i < 2 * _TPM, i, i - 2 * _TPM), 0, _TPM - 1)
    return (l, s // _TPH, s % _TPH)


def _whh_index(i):
    l = jnp.where(i < 3 * _TPM, 0, 1)
    s = jnp.clip(jnp.where(i < 3 * _TPM, i - _TPM, i - 3 * _TPM),
                 0, _TPM - 1)
    return (l, s // _TPH, s % _TPH)


def _full(shape):
    return pl.BlockSpec(shape, lambda i: tuple(0 for _ in shape))


def kernel(g_emb, w_emb, soft_emb, W_ih, W_hh, b_ih, b_hh):
    gum = _gumbel_table()
    stats, arch_row = pl.pallas_call(
        _body,
        grid=(_GRID,),
        in_specs=[
            _full((1, _H)),
            _full((_A, _H)),
            _full((_A, _H)),
            pl.BlockSpec((1, _TH, _TW), _wih_index),
            pl.BlockSpec((1, _TH, _TW), _whh_index),
            _full((2, 4 * _H)),
            _full((2, 4 * _H)),
            _full((_STEPS, _A)),
        ],
        out_specs=[
            _full((2, _STEPS)),
            _full((1, _STEPS)),
        ],
        out_shape=[
            jax.ShapeDtypeStruct((2, _STEPS), _F32),
            jax.ShapeDtypeStruct((1, _STEPS), jnp.int32),
        ],
        scratch_shapes=[
            pltpu.VMEM((2 * _H, 4 * _H), _BF16),
            pltpu.VMEM((4 * _H, 4 * _H), _BF16),
            pltpu.VMEM((9, 4 * _H), _F32),
            pltpu.VMEM((_STEPS, _A), _F32),
        ],
        compiler_params=pltpu.CompilerParams(
            dimension_semantics=("arbitrary",),
            vmem_limit_bytes=63 * 1024 * 1024),
    )(g_emb, w_emb, soft_emb.T, W_ih, W_hh, b_ih, b_hh, gum)
    return stats, arch_row[0]


# speedup vs baseline: 2.0835x; 1.0446x over previous
import jax
import jax.numpy as jnp
import numpy as np
from jax.experimental import pallas as pl
from jax.experimental.pallas import tpu as pltpu

_STEPS = 24
_A = 8
_H = 1024
_TW = 256
_TH = 2 * _H
_TPH = _H // _TW
_TPM = 2 * _TPH
_GRID = 4 * _TPM + 1
_F32 = jnp.float32
_BF16 = jnp.bfloat16
_HI = jax.lax.Precision.HIGHEST


def _gumbel_table():
    skey = jax.random.key(42)
    rows = [
        jax.random.gumbel(jax.random.fold_in(skey, s), (1, _A), _F32)
        for s in range(_STEPS)
    ]
    return jnp.concatenate(rows, axis=0)


def _body(g_ref, emb_ref, softt_ref, wih_ref, whh_ref, bih_ref, bhh_ref,
          gum_ref, stats_ref, arch_ref, w0s, w1s, es, ls):
    H = _H
    i = pl.program_id(0)

    def prologue(start, half, work):
        @pl.when((i >= start) & (i < start + _TPH))
        def _():
            j = i - start
            work(j * _TW, j, half * _TH)

    def e_work(ref):
        def work(row_base, j, col0):
            rows_tile = jnp.concatenate(
                [g_ref[:, pl.ds(row_base, _TW)],
                 emb_ref[:, pl.ds(row_base, _TW)]], axis=0)
            part = jax.lax.dot_general(
                ref[0], rows_tile.T, (((1,), (0,)), ((), ())),
                preferred_element_type=_F32, precision=_HI)
            prev = jnp.where(j == 0, jnp.zeros((9, _TH), _F32),
                             es[:, col0:col0 + _TH])
            es[:, col0:col0 + _TH] = prev + part.T
        return work

    def split_work(ref, out_ref, base):
        def work(row_base, j, col0):
            bt = ref[0].T
            hi = bt.astype(_BF16)
            lo = (bt - hi.astype(_F32)).astype(_BF16)
            out_ref[pl.ds(base + row_base, _TW), col0:col0 + _TH] = hi
            out_ref[pl.ds(base + H + row_base, _TW), col0:col0 + _TH] = lo
        return work

    for half in (0, 1):
        prologue(0 * _TPM + half * _TPH, half, e_work(wih_ref))
        prologue(1 * _TPM + half * _TPH, half, split_work(whh_ref, w0s, 0))
        prologue(2 * _TPM + half * _TPH, half, split_work(wih_ref, w1s, 0))
        prologue(3 * _TPM + half * _TPH, half,
                 split_work(whh_ref, w1s, 2 * H))

    @pl.when(i == _GRID - 1)
    def _():
        iota_a = jax.lax.broadcasted_iota(jnp.int32, (1, _A), 1)
        iota_t = jax.lax.broadcasted_iota(jnp.int32, (1, _STEPS), 1)
        b0 = bih_ref[0:1, :] + bhh_ref[0:1, :]
        b1 = bih_ref[1:2, :] + bhh_ref[1:2, :]
        soft = softt_ref[...].T

        def hilo2(x):
            hi = x.astype(_BF16)
            lo = (x - hi.astype(_F32)).astype(_BF16)
            return jnp.concatenate([
                jnp.concatenate([hi, hi], axis=1),
                jnp.concatenate([lo, jnp.zeros((1, H), _BF16)], axis=1)],
                axis=0)

        def cell(gates, c):
            i_g = gates[:, 0:H]
            f_g = gates[:, H:2 * H]
            g_g = gates[:, 2 * H:3 * H]
            o_g = gates[:, 3 * H:4 * H]
            c_new = (jax.nn.sigmoid(f_g) * c
                     + jax.nn.sigmoid(i_g) * jnp.tanh(g_g))
            h_new = jax.nn.sigmoid(o_g) * jnp.tanh(c_new)
            return h_new, c_new

        def bdot(act, wv):
            r = jax.lax.dot_general(
                act, wv, (((1,), (0,)), ((), ())),
                preferred_element_type=_F32)
            return r[0:1, :] + r[1:2, :]

        def step_fn(t, carry):
            x0e, d0, c0, d1h, c1, act_row = carry
            g0 = x0e + d0 + b0
            h0n, c0n = cell(g0, c0)
            a0n = hilo2(h0n)
            d0n = bdot(a0n, w0s[...])
            d1x = bdot(a0n, w1s[0:2 * H, :])
            g1 = d1x + d1h + b1
            h1n, c1n = cell(g1, c1)
            d1hn = bdot(hilo2(h1n), w1s[2 * H:4 * H, :])
            logits = jax.lax.dot_general(
                h1n, soft, (((1,), (0,)), ((), ())),
                preferred_element_type=_F32, precision=_HI)
            ls[pl.ds(t, 1), :] = logits
            z = logits + gum_ref[pl.ds(t, 1), :]
            a = jnp.min(
                jnp.where(z >= jnp.max(z), iota_a, _A)).astype(jnp.int32)
            x0e_next = es[pl.ds(a + 1, 1), :]
            act_row = jnp.where(iota_t == t, a, act_row)
            return (x0e_next, d0n, c0n, d1hn, c1n, act_row)

        zvec = jnp.zeros((1, 4 * H), _F32)
        init = (es[0:1, :], zvec, jnp.zeros((1, H), _F32),
                zvec, jnp.zeros((1, H), _F32),
                jnp.zeros((1, _STEPS), jnp.int32))
        carry = jax.lax.fori_loop(0, _STEPS, step_fn, init)
        act_row = carry[5]

        L = ls[...]
        m = jnp.max(L, axis=1, keepdims=True)
        logp = L - (m + jnp.log(jnp.sum(jnp.exp(L - m), axis=1,
                                        keepdims=True)))
        ent_col = -jnp.sum(jnp.exp(logp) * logp, axis=1, keepdims=True)
        oh = (jax.lax.broadcasted_iota(jnp.int32, (_STEPS, _A), 1)
              == act_row.T)
        lp_col = jnp.sum(jnp.where(oh, logp, 0.0), axis=1, keepdims=True)
        stats_ref[0:1, :] = lp_col.T
        stats_ref[1:2, :] = ent_col.T
        arch_ref[...] = act_row


def _wih_index(i):
    l = jnp.where(i < 2 * _TPM, 0, 1)
    s = jnp.clip(jnp.where(i < 2 * _TPM, i, i - 2 * _TPM), 0, _TPM - 1)
    return (l, s // _TPH, s % _TPH)


def _whh_index(i):
    l = jnp.where(i < 3 * _TPM, 0, 1)
    s = jnp.clip(jnp.where(i < 3 * _TPM, i - _TPM, i - 3 * _TPM),
                 0, _TPM - 1)
    return (l, s // _TPH, s % _TPH)


def _full(shape):
    return pl.BlockSpec(shape, lambda i: tuple(0 for _ in shape))


def kernel(g_emb, w_emb, soft_emb, W_ih, W_hh, b_ih, b_hh):
    gum = _gumbel_table()
    stats, arch_row = pl.pallas_call(
        _body,
        grid=(_GRID,),
        in_specs=[
            _full((1, _H)),
            _full((_A, _H)),
            _full((_A, _H)),
            pl.BlockSpec((1, _TH, _TW), _wih_index),
            pl.BlockSpec((1, _TH, _TW), _whh_index),
            _full((2, 4 * _H)),
            _full((2, 4 * _H)),
            _full((_STEPS, _A)),
        ],
        out_specs=[
            _full((2, _STEPS)),
            _full((1, _STEPS)),
        ],
        out_shape=[
            jax.ShapeDtypeStruct((2, _STEPS), _F32),
            jax.ShapeDtypeStruct((1, _STEPS), jnp.int32),
        ],
        scratch_shapes=[
            pltpu.VMEM((2 * _H, 4 * _H), _BF16),
            pltpu.VMEM((4 * _H, 4 * _H), _BF16),
            pltpu.VMEM((9, 4 * _H), _F32),
            pltpu.VMEM((_STEPS, _A), _F32),
        ],
        compiler_params=pltpu.CompilerParams(
            dimension_semantics=("arbitrary",),
            vmem_limit_bytes=63 * 1024 * 1024),
    )(g_emb, w_emb, soft_emb.T, W_ih, W_hh, b_ih, b_hh, gum)
    return stats, arch_row[0]


# speedup vs baseline: 2.0843x; 1.0004x over previous
import jax
import jax.numpy as jnp
import numpy as np
from jax.experimental import pallas as pl
from jax.experimental.pallas import tpu as pltpu

_STEPS = 24
_A = 8
_H = 1024
_TW = 256
_TH = 2 * _H
_TPH = _H // _TW
_TPM = 2 * _TPH
_GRID = 4 * _TPM + 1
_F32 = jnp.float32
_BF16 = jnp.bfloat16
_HI = jax.lax.Precision.HIGHEST


def _gumbel_table():
    skey = jax.random.key(42)
    rows = [
        jax.random.gumbel(jax.random.fold_in(skey, s), (1, _A), _F32)
        for s in range(_STEPS)
    ]
    return jnp.concatenate(rows, axis=0)


def _body(g_ref, emb_ref, softt_ref, wih_ref, whh_ref, bih_ref, bhh_ref,
          gum_ref, stats_ref, arch_ref, wa, wb, es, ls):
    H = _H
    i = pl.program_id(0)

    def prologue(start, half, work):
        @pl.when((i >= start) & (i < start + _TPH))
        def _():
            j = i - start
            work(j * _TW, j, half * _TH)

    def e_work(ref):
        def work(row_base, j, col0):
            rows_tile = jnp.concatenate(
                [g_ref[:, pl.ds(row_base, _TW)],
                 emb_ref[:, pl.ds(row_base, _TW)]], axis=0)
            part = jax.lax.dot_general(
                ref[0], rows_tile.T, (((1,), (0,)), ((), ())),
                preferred_element_type=_F32, precision=_HI)
            prev = jnp.where(j == 0, jnp.zeros((9, _TH), _F32),
                             es[:, col0:col0 + _TH])
            es[:, col0:col0 + _TH] = prev + part.T
        return work

    def split_work(ref, out_ref, coloff):
        def work(row_base, j, col0):
            bt = ref[0].T
            hi = bt.astype(_BF16)
            lo = (bt - hi.astype(_F32)).astype(_BF16)
            c0 = coloff + col0
            out_ref[pl.ds(row_base, _TW), c0:c0 + _TH] = hi
            out_ref[pl.ds(H + row_base, _TW), c0:c0 + _TH] = lo
        return work

    for half in (0, 1):
        prologue(0 * _TPM + half * _TPH, half, e_work(wih_ref))
        prologue(1 * _TPM + half * _TPH, half, split_work(whh_ref, wa, 0))
        prologue(2 * _TPM + half * _TPH, half,
                 split_work(wih_ref, wa, 4 * H))
        prologue(3 * _TPM + half * _TPH, half, split_work(whh_ref, wb, 0))

    @pl.when(i == _GRID - 1)
    def _():
        iota_a = jax.lax.broadcasted_iota(jnp.int32, (1, _A), 1)
        iota_t = jax.lax.broadcasted_iota(jnp.int32, (1, _STEPS), 1)
        b0 = bih_ref[0:1, :] + bhh_ref[0:1, :]
        b1 = bih_ref[1:2, :] + bhh_ref[1:2, :]
        soft = softt_ref[...].T

        def hilo2(x):
            hi = x.astype(_BF16)
            lo = (x - hi.astype(_F32)).astype(_BF16)
            return jnp.concatenate([
                jnp.concatenate([hi, hi], axis=1),
                jnp.concatenate([lo, jnp.zeros((1, H), _BF16)], axis=1)],
                axis=0)

        def cell(gates, c):
            i_g = gates[:, 0:H]
            f_g = gates[:, H:2 * H]
            g_g = gates[:, 2 * H:3 * H]
            o_g = gates[:, 3 * H:4 * H]
            c_new = (jax.nn.sigmoid(f_g) * c
                     + jax.nn.sigmoid(i_g) * jnp.tanh(g_g))
            h_new = jax.nn.sigmoid(o_g) * jnp.tanh(c_new)
            return h_new, c_new

        def bdot(act, wv):
            r = jax.lax.dot_general(
                act, wv, (((1,), (0,)), ((), ())),
                preferred_element_type=_F32)
            return r[0:1, :] + r[1:2, :]

        def step_fn(t, carry):
            x0e, d0, c0, d1h, c1, act_row = carry
            g0 = x0e + d0 + b0
            h0n, c0n = cell(g0, c0)
            a0n = hilo2(h0n)
            ra = jax.lax.dot_general(
                a0n, wa[...], (((1,), (0,)), ((), ())),
                preferred_element_type=_F32)
            d0n = ra[0:1, 0:4 * H] + ra[1:2, 0:4 * H]
            d1x = ra[0:1, 4 * H:8 * H] + ra[1:2, 4 * H:8 * H]
            g1 = d1x + d1h + b1
            h1n, c1n = cell(g1, c1)
            d1hn = bdot(hilo2(h1n), wb[...])
            logits = jax.lax.dot_general(
                h1n, soft, (((1,), (0,)), ((), ())),
                preferred_element_type=_F32, precision=_HI)
            ls[pl.ds(t, 1), :] = logits
            z = logits + gum_ref[pl.ds(t, 1), :]
            a = jnp.min(
                jnp.where(z >= jnp.max(z), iota_a, _A)).astype(jnp.int32)
            x0e_next = es[pl.ds(a + 1, 1), :]
            act_row = jnp.where(iota_t == t, a, act_row)
            return (x0e_next, d0n, c0n, d1hn, c1n, act_row)

        zvec = jnp.zeros((1, 4 * H), _F32)
        init = (es[0:1, :], zvec, jnp.zeros((1, H), _F32),
                zvec, jnp.zeros((1, H), _F32),
                jnp.zeros((1, _STEPS), jnp.int32))
        carry = jax.lax.fori_loop(0, _STEPS, step_fn, init)
        act_row = carry[5]

        L = ls[...]
        m = jnp.max(L, axis=1, keepdims=True)
        logp = L - (m + jnp.log(jnp.sum(jnp.exp(L - m), axis=1,
                                        keepdims=True)))
        ent_col = -jnp.sum(jnp.exp(logp) * logp, axis=1, keepdims=True)
        oh = (jax.lax.broadcasted_iota(jnp.int32, (_STEPS, _A), 1)
              == act_row.T)
        lp_col = jnp.sum(jnp.where(oh, logp, 0.0), axis=1, keepdims=True)
        stats_ref[0:1, :] = lp_col.T
        stats_ref[1:2, :] = ent_col.T
        arch_ref[...] = act_row


def _wih_index(i):
    l = jnp.where(i < 2 * _TPM, 0, 1)
    s = jnp.clip(jnp.where(i < 2 * _TPM, i, i - 2 * _TPM), 0, _TPM - 1)
    return (l, s // _TPH, s % _TPH)


def _whh_index(i):
    l = jnp.where(i < 3 * _TPM, 0, 1)
    s = jnp.clip(jnp.where(i < 3 * _TPM, i - _TPM, i - 3 * _TPM),
                 0, _TPM - 1)
    return (l, s // _TPH, s % _TPH)


def _full(shape):
    return pl.BlockSpec(shape, lambda i: tuple(0 for _ in shape))


def kernel(g_emb, w_emb, soft_emb, W_ih, W_hh, b_ih, b_hh):
    gum = _gumbel_table()
    stats, arch_row = pl.pallas_call(
        _body,
        grid=(_GRID,),
        in_specs=[
            _full((1, _H)),
            _full((_A, _H)),
            _full((_A, _H)),
            pl.BlockSpec((1, _TH, _TW), _wih_index),
            pl.BlockSpec((1, _TH, _TW), _whh_index),
            _full((2, 4 * _H)),
            _full((2, 4 * _H)),
            _full((_STEPS, _A)),
        ],
        out_specs=[
            _full((2, _STEPS)),
            _full((1, _STEPS)),
        ],
        out_shape=[
            jax.ShapeDtypeStruct((2, _STEPS), _F32),
            jax.ShapeDtypeStruct((1, _STEPS), jnp.int32),
        ],
        scratch_shapes=[
            pltpu.VMEM((2 * _H, 8 * _H), _BF16),
            pltpu.VMEM((2 * _H, 4 * _H), _BF16),
            pltpu.VMEM((9, 4 * _H), _F32),
            pltpu.VMEM((_STEPS, _A), _F32),
        ],
        compiler_params=pltpu.CompilerParams(
            dimension_semantics=("arbitrary",),
            vmem_limit_bytes=63 * 1024 * 1024),
    )(g_emb, w_emb, soft_emb.T, W_ih, W_hh, b_ih, b_hh, gum)
    return stats, arch_row[0]
